# Initial kernel scaffold; baseline (speedup 1.0000x reference)
import jax
import jax.numpy as jnp
from jax.experimental import pallas as pl


def kernel(concept_embeddings, fusion_weights, emb_table, W, b):
    raise NotImplementedError("write your pallas kernel here")


# speedup vs baseline: 1.4101x; 1.4101x over previous
import functools

import jax
import jax.numpy as jnp
from jax import lax
from jax.experimental import pallas as pl
from jax.experimental.pallas import tpu as pltpu
from jax.experimental.pallas import tpu_sc as plsc

_IDX_CHUNK = 128


@functools.lru_cache(maxsize=None)
def _sc_gather(num_workers: int, n_chunks: int, n_rows: int, d: int):
    b_per_w = n_chunks * _IDX_CHUNK
    mesh = plsc.VectorSubcoreMesh(core_axis_name="c", subcore_axis_name="s")
    nc = 2

    @functools.partial(
        pl.kernel,
        out_type=jax.ShapeDtypeStruct((num_workers * b_per_w, d), jnp.float32),
        mesh=mesh,
        scratch_types=[
            pltpu.VMEM((n_chunks, _IDX_CHUNK), jnp.int32),
            pltpu.VMEM((b_per_w, d), jnp.float32),
            pltpu.SemaphoreType.DMA,
        ],
    )
    def gather(idx_hbm, table_hbm, out_hbm, idx_v, rows_v, sem):
        wid = lax.axis_index("s") * nc + lax.axis_index("c")
        pltpu.sync_copy(idx_hbm.at[wid], idx_v)
        copies = [
            pltpu.async_copy(
                table_hbm.at[idx_v.at[j]],
                rows_v.at[pl.ds(j * _IDX_CHUNK, _IDX_CHUNK)],
                sem,
            )
            for j in range(n_chunks)
        ]
        for c in copies:
            c.wait()
        pltpu.sync_copy(rows_v, out_hbm.at[pl.ds(wid * b_per_w, b_per_w)])

    return gather


def _tc_fused(e, fw, w1t, w2t, b2d):
    bsz, d = e.shape
    f = w2t.shape[1]
    blk = 2048

    def body(e_ref, f_ref, w1_ref, w2_ref, b_ref, o_ref):
        acc = jnp.dot(e_ref[...], w1_ref[...], preferred_element_type=jnp.float32)
        acc = acc + jnp.dot(f_ref[...], w2_ref[...], preferred_element_type=jnp.float32)
        o_ref[...] = acc + b_ref[...]

    return pl.pallas_call(
        body,
        grid=(bsz // blk,),
        in_specs=[
            pl.BlockSpec((blk, d), lambda i: (i, 0)),
            pl.BlockSpec((blk, fw.shape[1]), lambda i: (i, 0)),
            pl.BlockSpec(w1t.shape, lambda i: (0, 0)),
            pl.BlockSpec(w2t.shape, lambda i: (0, 0)),
            pl.BlockSpec((1, f), lambda i: (0, 0)),
        ],
        out_specs=pl.BlockSpec((blk, f), lambda i: (i, 0)),
        out_shape=jax.ShapeDtypeStruct((bsz, f), jnp.float32),
    )(e, fw, w1t, w2t, b2d)


def kernel(concept_embeddings, fusion_weights, emb_table, W, b):
    bsz = concept_embeddings.shape[0]
    n_rows, d = emb_table.shape
    num_workers = 32
    b_per_w = bsz // num_workers
    n_chunks = b_per_w // _IDX_CHUNK

    idx = concept_embeddings.astype(jnp.int32).reshape(
        num_workers, n_chunks, _IDX_CHUNK
    )
    e = _sc_gather(num_workers, n_chunks, n_rows, d)(idx, emb_table)

    w1t = W[:, :d].T
    w2t = W[:, d:].T
    return _tc_fused(e, fusion_weights, w1t, w2t, b.reshape(1, -1))
